# Initial kernel scaffold; baseline (speedup 1.0000x reference)
import jax
import jax.numpy as jnp
from jax.experimental import pallas as pl


def kernel(x, gain, bias, log_mix, log_scale):
    raise NotImplementedError("write your pallas kernel here")


# speedup vs baseline: 20.1771x; 20.1771x over previous
import functools

import jax
import jax.numpy as jnp
from jax.experimental import pallas as pl

_K = 8
_NEG = -1e30


def _fused_kernel(x_ref, gain_ref, bias_ref, lm_ref, ls_ref, out_ref, *, blk_t):
    i = pl.program_id(1)
    xa = x_ref[0]
    t_total = xa.shape[0]

    n2 = jnp.sum(xa * xa, axis=1, keepdims=True)
    xn = xa / (jnp.sqrt(n2) + 1e-8)

    row0 = i * blk_t
    x_rows = x_ref[0, pl.ds(row0, blk_t), :]
    n2r = jnp.sum(x_rows * x_rows, axis=1, keepdims=True)
    xn_rows = x_rows / (jnp.sqrt(n2r) + 1e-8)

    sim = jax.lax.dot_general(
        xn_rows, xn, (((1,), (1,)), ((), ())),
        preferred_element_type=jnp.float32)

    cols = jax.lax.broadcasted_iota(jnp.int32, (blk_t, t_total), 1)
    rows = row0 + jax.lax.broadcasted_iota(jnp.int32, (blk_t, t_total), 0)
    causal = cols <= rows

    work = jnp.where(causal, sim, _NEG)
    thresh = None
    for _ in range(_K):
        thresh = jnp.max(work, axis=1, keepdims=True)
        work = jnp.where(work == thresh, _NEG, work)

    adj = jnp.where(causal & (sim >= thresh), 1.0, 0.0)
    deg = jnp.sum(adj, axis=1, keepdims=True)

    msg = jax.lax.dot_general(
        adj, xa, (((1,), (0,)), ((), ())),
        preferred_element_type=jnp.float32)
    msg = msg / jnp.maximum(deg, 1.0)

    mix = jax.nn.sigmoid(lm_ref[0, 0])
    scale = jax.nn.softplus(ls_ref[0, 0]) + 0.01

    blended = mix * x_rows + (1.0 - mix) * msg
    y = blended * gain_ref[0][None, :] + bias_ref[0][None, :]
    gelu = 0.5 * y * (1.0 + jax.lax.erf(y * (2.0 ** -0.5)))
    out_ref[0] = gelu * scale


def kernel(x, gain, bias, log_mix, log_scale):
    B, T, D = x.shape
    blk_t = min(256, T)
    grid = (B, T // blk_t)

    fn = functools.partial(_fused_kernel, blk_t=blk_t)
    return pl.pallas_call(
        fn,
        grid=grid,
        in_specs=[
            pl.BlockSpec((1, T, D), lambda b, i: (b, 0, 0)),
            pl.BlockSpec((1, D), lambda b, i: (0, 0)),
            pl.BlockSpec((1, D), lambda b, i: (0, 0)),
            pl.BlockSpec((1, 1), lambda b, i: (0, 0)),
            pl.BlockSpec((1, 1), lambda b, i: (0, 0)),
        ],
        out_specs=pl.BlockSpec((1, blk_t, D), lambda b, i: (b, i, 0)),
        out_shape=jax.ShapeDtypeStruct((B, T, D), x.dtype),
    )(x, gain.reshape(1, D), bias.reshape(1, D),
      log_mix.reshape(1, 1), log_scale.reshape(1, 1))


# speedup vs baseline: 22.4466x; 1.1125x over previous
import functools

import jax
import jax.numpy as jnp
from jax.experimental import pallas as pl
from jax.experimental.pallas import tpu as pltpu

_K = 8
_NEG = -1e30


def _fused_kernel(x_ref, gain_ref, bias_ref, lm_ref, ls_ref, out_ref, xn_ref,
                  *, blk_t):
    i = pl.program_id(1)
    t_total = x_ref.shape[1]

    @pl.when(i == 0)
    def _normalize():
        xa_full = x_ref[0]
        n2 = jnp.sum(xa_full * xa_full, axis=1, keepdims=True)
        xn_ref[...] = xa_full / (jnp.sqrt(n2) + 1e-8)

    row0 = i * blk_t
    xn_rows = xn_ref[pl.ds(row0, blk_t), :]

    sim = jax.lax.dot_general(
        xn_rows, xn_ref[...], (((1,), (1,)), ((), ())),
        preferred_element_type=jnp.float32)

    cols = jax.lax.broadcasted_iota(jnp.int32, (blk_t, t_total), 1)
    rows = row0 + jax.lax.broadcasted_iota(jnp.int32, (blk_t, t_total), 0)
    w = jnp.where(cols <= rows, sim, _NEG)

    m = jnp.max(w, axis=1, keepdims=True)
    for _ in range(_K - 1):
        m = jnp.max(jnp.where(w < m, w, _NEG), axis=1, keepdims=True)
    thresh = jnp.maximum(m, -2.0)

    adj = jnp.where(w >= thresh, 1.0, 0.0)
    deg = jnp.sum(adj, axis=1, keepdims=True)

    msg = jax.lax.dot_general(
        adj, x_ref[0], (((1,), (0,)), ((), ())),
        preferred_element_type=jnp.float32)
    msg = msg / jnp.maximum(deg, 1.0)

    mix = jax.nn.sigmoid(lm_ref[0, 0])
    scale = jax.nn.softplus(ls_ref[0, 0]) + 0.01

    x_rows = x_ref[0, pl.ds(row0, blk_t), :]
    blended = mix * x_rows + (1.0 - mix) * msg
    y = blended * gain_ref[0][None, :] + bias_ref[0][None, :]
    gelu = 0.5 * y * (1.0 + jax.lax.erf(y * (2.0 ** -0.5)))
    out_ref[0] = gelu * scale


def kernel(x, gain, bias, log_mix, log_scale):
    B, T, D = x.shape
    blk_t = min(256, T)
    grid = (B, T // blk_t)

    fn = functools.partial(_fused_kernel, blk_t=blk_t)
    return pl.pallas_call(
        fn,
        grid=grid,
        in_specs=[
            pl.BlockSpec((1, T, D), lambda b, i: (b, 0, 0)),
            pl.BlockSpec((1, D), lambda b, i: (0, 0)),
            pl.BlockSpec((1, D), lambda b, i: (0, 0)),
            pl.BlockSpec((1, 1), lambda b, i: (0, 0)),
            pl.BlockSpec((1, 1), lambda b, i: (0, 0)),
        ],
        out_specs=pl.BlockSpec((1, blk_t, D), lambda b, i: (b, i, 0)),
        out_shape=jax.ShapeDtypeStruct((B, T, D), x.dtype),
        scratch_shapes=[pltpu.VMEM((T, D), jnp.float32)],
    )(x, gain.reshape(1, D), bias.reshape(1, D),
      log_mix.reshape(1, 1), log_scale.reshape(1, 1))
